# baseline (speedup 1.0000x reference)
import functools

import jax
import jax.numpy as jnp
from jax import lax
from jax.experimental import pallas as pl
from jax.experimental.pallas import tpu as pltpu
from jax.experimental.pallas import tpu_sc as plsc

BATCH = 4096
HIST = 200
DIM = 64
N = BATCH * HIST
NC, NS = 2, 16
NW = NC * NS
PER_W = N // NW
CHUNK = 128
G = PER_W // CHUNK


def _body(idx_hbm, table_hbm, out_hbm, idx_v, rows_v, sem):
    wid = lax.axis_index("s") * NC + lax.axis_index("c")
    pltpu.sync_copy(idx_hbm.at[wid], idx_v)
    base = wid * PER_W

    def chunk(j, carry):
        pltpu.async_copy(table_hbm.at[idx_v.at[j]], rows_v, sem).wait()
        pltpu.sync_copy(rows_v, out_hbm.at[pl.ds(base + j * CHUNK, CHUNK)])
        return carry

    lax.fori_loop(0, G, chunk, 0)


@jax.jit
def _gather(idx3, weight):
    mesh = plsc.VectorSubcoreMesh(core_axis_name="c", subcore_axis_name="s")
    return pl.kernel(
        _body,
        out_type=jax.ShapeDtypeStruct((N, DIM), jnp.float32),
        mesh=mesh,
        scratch_types=[
            pltpu.VMEM((G, CHUNK), jnp.int32),
            pltpu.VMEM((CHUNK, DIM), jnp.float32),
            pltpu.SemaphoreType.DMA,
        ],
        compiler_params=pltpu.CompilerParams(use_tc_tiling_on_sc=False),
    )(idx3, weight)


def kernel(input_, weight):
    idx3 = input_.astype(jnp.int32).reshape(NW, G, CHUNK)
    out = _gather(idx3, weight)
    return out.reshape(BATCH, HIST, DIM)


# speedup vs baseline: 1.1143x; 1.1143x over previous
import functools

import jax
import jax.numpy as jnp
from jax import lax
from jax.experimental import pallas as pl
from jax.experimental.pallas import tpu as pltpu
from jax.experimental.pallas import tpu_sc as plsc

BATCH = 4096
HIST = 200
DIM = 64
N = BATCH * HIST
NC, NS = 2, 16
NW = NC * NS
PER_W = N // NW
CHUNK = 128
G = PER_W // CHUNK


NBUF = 8


def _body(idx_hbm, table_hbm, out_hbm, idx_v, rows_v, sem_g, sem_s):
    wid = lax.axis_index("s") * NC + lax.axis_index("c")
    pltpu.sync_copy(idx_hbm.at[wid], idx_v)
    base = wid * PER_W

    def gather_start(j, b):
        pltpu.async_copy(table_hbm.at[idx_v.at[j]], rows_v.at[b], sem_g)

    def gather_wait(b):
        pltpu.make_async_copy(
            table_hbm.at[idx_v.at[0]], rows_v.at[b], sem_g).wait()

    def store_start(j, b):
        pltpu.async_copy(
            rows_v.at[b], out_hbm.at[pl.ds(base + j * CHUNK, CHUNK)], sem_s)

    def store_wait(b):
        pltpu.make_async_copy(
            rows_v.at[b], out_hbm.at[pl.ds(base, CHUNK)], sem_s).wait()

    for b in range(NBUF):
        gather_start(b, b)

    def outer(o, carry):
        for b in range(NBUF):
            j = o * NBUF + b
            gather_wait(b)
            store_start(j, b)

            @pl.when(j + NBUF < G)
            def _():
                store_wait(b)
                gather_start(j + NBUF, b)
        return carry

    lax.fori_loop(0, G // NBUF, outer, 0)
    for b in range(NBUF):
        store_wait(b)


@jax.jit
def _gather(idx3, weight):
    mesh = plsc.VectorSubcoreMesh(core_axis_name="c", subcore_axis_name="s")
    return pl.kernel(
        _body,
        out_type=jax.ShapeDtypeStruct((N, DIM), jnp.float32),
        mesh=mesh,
        scratch_types=[
            pltpu.VMEM((G, CHUNK), jnp.int32),
            pltpu.VMEM((NBUF, CHUNK, DIM), jnp.float32),
            pltpu.SemaphoreType.DMA,
            pltpu.SemaphoreType.DMA,
        ],
        compiler_params=pltpu.CompilerParams(use_tc_tiling_on_sc=False),
    )(idx3, weight)


def kernel(input_, weight):
    idx3 = input_.astype(jnp.int32).reshape(NW, G, CHUNK)
    out = _gather(idx3, weight)
    return out.reshape(BATCH, HIST, DIM)


# speedup vs baseline: 1.1159x; 1.0015x over previous
import functools

import jax
import jax.numpy as jnp
from jax import lax
from jax.experimental import pallas as pl
from jax.experimental.pallas import tpu as pltpu
from jax.experimental.pallas import tpu_sc as plsc

BATCH = 4096
HIST = 200
DIM = 64
N = BATCH * HIST
NC, NS = 2, 16
NW = NC * NS
PER_W = N // NW
CHUNK = 256
G = PER_W // CHUNK


NBUF = 4


def _body(idx_hbm, table_hbm, out_hbm, idx_v, rows_v, sem_g, sem_s):
    wid = lax.axis_index("s") * NC + lax.axis_index("c")
    pltpu.sync_copy(idx_hbm.at[wid], idx_v)
    base = wid * PER_W

    def gather_start(j, b):
        pltpu.async_copy(table_hbm.at[idx_v.at[j]], rows_v.at[b], sem_g)

    def gather_wait(b):
        pltpu.make_async_copy(
            table_hbm.at[idx_v.at[0]], rows_v.at[b], sem_g).wait()

    def store_start(j, b):
        pltpu.async_copy(
            rows_v.at[b], out_hbm.at[pl.ds(base + j * CHUNK, CHUNK)], sem_s)

    def store_wait(b):
        pltpu.make_async_copy(
            rows_v.at[b], out_hbm.at[pl.ds(base, CHUNK)], sem_s).wait()

    for b in range(NBUF):
        gather_start(b, b)

    def outer(o, carry):
        for b in range(NBUF):
            j = o * NBUF + b
            gather_wait(b)
            store_start(j, b)

            @pl.when(j + NBUF < G)
            def _():
                store_wait(b)
                gather_start(j + NBUF, b)
        return carry

    lax.fori_loop(0, G // NBUF, outer, 0)
    for b in range(NBUF):
        store_wait(b)


@jax.jit
def _gather(idx3, weight):
    mesh = plsc.VectorSubcoreMesh(core_axis_name="c", subcore_axis_name="s")
    return pl.kernel(
        _body,
        out_type=jax.ShapeDtypeStruct((N, DIM), jnp.float32),
        mesh=mesh,
        scratch_types=[
            pltpu.VMEM((G, CHUNK), jnp.int32),
            pltpu.VMEM((NBUF, CHUNK, DIM), jnp.float32),
            pltpu.SemaphoreType.DMA,
            pltpu.SemaphoreType.DMA,
        ],
        compiler_params=pltpu.CompilerParams(use_tc_tiling_on_sc=False),
    )(idx3, weight)


def kernel(input_, weight):
    idx3 = input_.astype(jnp.int32).reshape(NW, G, CHUNK)
    out = _gather(idx3, weight)
    return out.reshape(BATCH, HIST, DIM)
